# baseline (speedup 1.0000x reference)
import jax
import jax.numpy as jnp
from jax import lax
from jax.experimental import pallas as pl
from jax.experimental.pallas import tpu as pltpu
from jax.experimental.pallas import tpu_sc as plsc

_B, _F_OBS, _F_PRED, _J, _D = 1024, 20, 50, 24, 64
_F = _F_OBS + _F_PRED
_L = 16
_DP = 128
_NSEG = _DP // _L

_PAD_OBS = 4
_N_OBS = _PAD_OBS + _F_OBS
_N_PRED = 56
_N_TAB = _N_OBS + _N_PRED

_MAGIC = 0x5F3759DF

_GATHER_DNUMS = lax.GatherDimensionNumbers(
    offset_dims=(), collapsed_slice_dims=(0,), start_index_map=(0,))


def _renorm_rows(rows_v, row_lo, row_hi):
    iot = lax.iota(jnp.int32, _L)
    for r in range(row_lo, row_hi):
        segs = [rows_v[r, pl.ds(_L * j, _L)] for j in range(_NSEG)]
        acc = segs[0] * segs[0]
        for t in segs[1:]:
            acc = acc + t * t
        for k in (8, 4, 2, 1):
            perm = jnp.bitwise_xor(iot, k)[:, None]
            acc = acc + lax.gather(
                acc, perm, _GATHER_DNUMS, slice_sizes=(1,),
                mode=lax.GatherScatterMode.PROMISE_IN_BOUNDS)
        x = jnp.maximum(acc, 1e-14)
        i = lax.bitcast_convert_type(x, jnp.int32)
        i = jnp.full((_L,), _MAGIC, jnp.int32) - lax.shift_right_arithmetic(i, 1)
        y = lax.bitcast_convert_type(i, jnp.float32)
        for _ in range(3):
            y = y * (1.5 - 0.5 * x * y * y)
        scale = jnp.minimum(1.0, y)
        for j in range(_NSEG):
            rows_v[r, pl.ds(_L * j, _L)] = segs[j] * scale


def _sc_table_body(emb_obs_hbm, emb_pred_hbm, idx_obs_hbm, idx_pred_hbm,
                   table_hbm, idx_obs_v, idx_pred_v, rows_obs_v, rows_pred_v,
                   sem):
    c = lax.axis_index("c")
    s = lax.axis_index("s")

    @pl.when((c == 0) & (s == 0))
    def _():
        pltpu.sync_copy(idx_obs_hbm, idx_obs_v)
        pltpu.async_copy(emb_obs_hbm.at[idx_obs_v], rows_obs_v, sem).wait()
        _renorm_rows(rows_obs_v, _PAD_OBS, _N_OBS)
        pltpu.sync_copy(rows_obs_v, table_hbm.at[pl.ds(0, _N_OBS), :])

    @pl.when((c == 0) & (s == 1))
    def _():
        pltpu.sync_copy(idx_pred_hbm, idx_pred_v)
        pltpu.async_copy(emb_pred_hbm.at[idx_pred_v], rows_pred_v, sem).wait()
        _renorm_rows(rows_pred_v, 0, _F_PRED)
        pltpu.sync_copy(rows_pred_v, table_hbm.at[pl.ds(_N_OBS, _N_PRED), :])


_sc_table = pl.kernel(
    _sc_table_body,
    out_type=jax.ShapeDtypeStruct((_N_TAB, _DP), jnp.float32),
    mesh=plsc.VectorSubcoreMesh(core_axis_name="c", subcore_axis_name="s"),
    scratch_types=[
        pltpu.VMEM((_N_OBS,), jnp.int32),
        pltpu.VMEM((_N_PRED,), jnp.int32),
        pltpu.VMEM((_N_OBS, _DP), jnp.float32),
        pltpu.VMEM((_N_PRED, _DP), jnp.float32),
        pltpu.SemaphoreType.DMA,
    ],
)


def _add_body(x_ref, rows_ref, out_ref):
    out_ref[...] = x_ref[...] + rows_ref[...][None, :, None, :]


_BB = 8


def _add_rows(x, rows):
    return pl.pallas_call(
        _add_body,
        grid=(_B // _BB,),
        in_specs=[
            pl.BlockSpec((_BB, _F, _J, _D), lambda i: (i, 0, 0, 0)),
            pl.BlockSpec((_F, _D), lambda i: (0, 0)),
        ],
        out_specs=pl.BlockSpec((_BB, _F, _J, _D), lambda i: (i, 0, 0, 0)),
        out_shape=jax.ShapeDtypeStruct((_B, _F, _J, _D), jnp.float32),
    )(x, rows)


def kernel(x, emb_obs, emb_pred, in_F, out_F):
    idx_obs = (in_F - 1) - jnp.arange(_F_OBS, dtype=jnp.int32)
    idx_pred = jnp.arange(_F_PRED, dtype=jnp.int32) + (out_F - _F_PRED)
    idx_obs_p = jnp.concatenate(
        [jnp.zeros((_PAD_OBS,), jnp.int32), idx_obs.astype(jnp.int32)])
    idx_pred_p = jnp.concatenate(
        [idx_pred.astype(jnp.int32),
         jnp.zeros((_N_PRED - _F_PRED,), jnp.int32)])
    zo = jnp.zeros((emb_obs.shape[0], _DP - _D), jnp.float32)
    zp = jnp.zeros((emb_pred.shape[0], _DP - _D), jnp.float32)
    table = _sc_table(jnp.concatenate([emb_obs, zo], axis=1),
                      jnp.concatenate([emb_pred, zp], axis=1),
                      idx_obs_p, idx_pred_p)
    rows = table[_PAD_OBS:_PAD_OBS + _F, :_D]
    return _add_rows(x, rows)


# speedup vs baseline: 5.9264x; 5.9264x over previous
import jax
import jax.numpy as jnp
from jax import lax
from jax.experimental import pallas as pl
from jax.experimental.pallas import tpu as pltpu
from jax.experimental.pallas import tpu_sc as plsc

_B, _F_OBS, _F_PRED, _J, _D = 1024, 20, 50, 24, 64
_F = _F_OBS + _F_PRED
_L = 16
_DP = 128
_NSEG = _DP // _L

_PAD_OBS = 4
_N_OBS = _PAD_OBS + _F_OBS
_N_PRED = 56
_N_TAB = _N_OBS + _N_PRED

_MAGIC = 0x5F3759DF

_GATHER_DNUMS = lax.GatherDimensionNumbers(
    offset_dims=(), collapsed_slice_dims=(0,), start_index_map=(0,))


def _renorm_rows(rows_v, row_lo, row_hi):
    iot = lax.iota(jnp.int32, _L)
    for r in range(row_lo, row_hi):
        segs = [rows_v[r, pl.ds(_L * j, _L)] for j in range(_NSEG)]
        acc = segs[0] * segs[0]
        for t in segs[1:]:
            acc = acc + t * t
        for k in (8, 4, 2, 1):
            perm = jnp.bitwise_xor(iot, k)[:, None]
            acc = acc + lax.gather(
                acc, perm, _GATHER_DNUMS, slice_sizes=(1,),
                mode=lax.GatherScatterMode.PROMISE_IN_BOUNDS)
        x = jnp.maximum(acc, 1e-14)
        i = lax.bitcast_convert_type(x, jnp.int32)
        i = jnp.full((_L,), _MAGIC, jnp.int32) - lax.shift_right_arithmetic(i, 1)
        y = lax.bitcast_convert_type(i, jnp.float32)
        for _ in range(3):
            y = y * (1.5 - 0.5 * x * y * y)
        scale = jnp.minimum(1.0, y)
        for j in range(_NSEG):
            rows_v[r, pl.ds(_L * j, _L)] = segs[j] * scale


def _sc_table_body(emb_obs_hbm, emb_pred_hbm, idx_obs_hbm, idx_pred_hbm,
                   table_hbm, idx_obs_v, idx_pred_v, rows_obs_v, rows_pred_v,
                   sem):
    c = lax.axis_index("c")
    s = lax.axis_index("s")

    @pl.when((c == 0) & (s == 0))
    def _():
        pltpu.sync_copy(idx_obs_hbm, idx_obs_v)
        pltpu.async_copy(emb_obs_hbm.at[idx_obs_v], rows_obs_v, sem).wait()
        _renorm_rows(rows_obs_v, _PAD_OBS, _N_OBS)
        pltpu.sync_copy(rows_obs_v, table_hbm.at[pl.ds(0, _N_OBS), :])

    @pl.when((c == 0) & (s == 1))
    def _():
        pltpu.sync_copy(idx_pred_hbm, idx_pred_v)
        pltpu.async_copy(emb_pred_hbm.at[idx_pred_v], rows_pred_v, sem).wait()
        _renorm_rows(rows_pred_v, 0, _F_PRED)
        pltpu.sync_copy(rows_pred_v, table_hbm.at[pl.ds(_N_OBS, _N_PRED), :])


_sc_table = pl.kernel(
    _sc_table_body,
    out_type=jax.ShapeDtypeStruct((_N_TAB, _DP), jnp.float32),
    mesh=plsc.VectorSubcoreMesh(core_axis_name="c", subcore_axis_name="s"),
    scratch_types=[
        pltpu.VMEM((_N_OBS,), jnp.int32),
        pltpu.VMEM((_N_PRED,), jnp.int32),
        pltpu.VMEM((_N_OBS, _DP), jnp.float32),
        pltpu.VMEM((_N_PRED, _DP), jnp.float32),
        pltpu.SemaphoreType.DMA,
    ],
)


def _add_body(x_ref, rows_ref, out_ref):
    out_ref[...] = x_ref[...] + rows_ref[...][:, :, :, None]


_BF = 2


def _add_rows(xt, rows3):
    return pl.pallas_call(
        _add_body,
        grid=(_F // _BF,),
        in_specs=[
            pl.BlockSpec((_BF, _J, _D, _B), lambda i: (i, 0, 0, 0)),
            pl.BlockSpec((_BF, 1, _D), lambda i: (i, 0, 0)),
        ],
        out_specs=pl.BlockSpec((_BF, _J, _D, _B), lambda i: (i, 0, 0, 0)),
        out_shape=jax.ShapeDtypeStruct((_F, _J, _D, _B), jnp.float32),
    )(xt, rows3)


def kernel(x, emb_obs, emb_pred, in_F, out_F):
    idx_obs = (in_F - 1) - jnp.arange(_F_OBS, dtype=jnp.int32)
    idx_pred = jnp.arange(_F_PRED, dtype=jnp.int32) + (out_F - _F_PRED)
    idx_obs_p = jnp.concatenate(
        [jnp.zeros((_PAD_OBS,), jnp.int32), idx_obs.astype(jnp.int32)])
    idx_pred_p = jnp.concatenate(
        [idx_pred.astype(jnp.int32),
         jnp.zeros((_N_PRED - _F_PRED,), jnp.int32)])
    zo = jnp.zeros((emb_obs.shape[0], _DP - _D), jnp.float32)
    zp = jnp.zeros((emb_pred.shape[0], _DP - _D), jnp.float32)
    table = _sc_table(jnp.concatenate([emb_obs, zo], axis=1),
                      jnp.concatenate([emb_pred, zp], axis=1),
                      idx_obs_p, idx_pred_p)
    rows = table[_PAD_OBS:_PAD_OBS + _F, :_D]
    xt = jnp.transpose(x, (1, 2, 3, 0))
    out_t = _add_rows(xt, rows.reshape(_F, 1, _D))
    return jnp.transpose(out_t, (3, 0, 1, 2))
